# baseline (speedup 1.0000x reference)
import functools
import jax
import jax.numpy as jnp
from jax import lax
from jax.experimental import pallas as pl
from jax.experimental.pallas import tpu as pltpu
from jax.experimental.pallas import tpu_sc as plsc

_N = 10000
_E = 320000
_NW = 32
_BKT = 320
_NP = _NW * _BKT
_CH = 6400
_FLUSH = 2048
_CAP = 322048
_B = 128

_mesh = functools.partial(
    plsc.VectorSubcoreMesh, core_axis_name="c", subcore_axis_name="s")


def _wid():
    return lax.axis_index("s") * 2 + lax.axis_index("c")


def _uv_body(x_ref, a_ref, b_ref, bias_ref, u_ref, v_ref):
    x = x_ref[...]
    u_ref[...] = jnp.dot(x, a_ref[...], preferred_element_type=jnp.float32) + bias_ref[...]
    v_ref[...] = jnp.dot(x, b_ref[...], preferred_element_type=jnp.float32)


def _uv_matmul(xp, A, B, b1, blk=512):
    NP, f = xp.shape
    d_h = A.shape[1]
    return pl.pallas_call(
        _uv_body,
        grid=(NP // blk,),
        in_specs=[
            pl.BlockSpec((blk, f), lambda i: (i, 0)),
            pl.BlockSpec((f, d_h), lambda i: (0, 0)),
            pl.BlockSpec((f, d_h), lambda i: (0, 0)),
            pl.BlockSpec((1, d_h), lambda i: (0, 0)),
        ],
        out_specs=[
            pl.BlockSpec((blk, d_h), lambda i: (i, 0)),
            pl.BlockSpec((blk, d_h), lambda i: (i, 0)),
        ],
        out_shape=[
            jax.ShapeDtypeStruct((NP, d_h), jnp.float32),
            jax.ShapeDtypeStruct((NP, d_h), jnp.float32),
        ],
    )(xp, A, B, b1.reshape(1, d_h))


def _edge_body(ms_ref, w2_ref, b2_ref, h_ref):
    m = jax.nn.relu(ms_ref[...])
    h_ref[...] = jnp.dot(m, w2_ref[...], preferred_element_type=jnp.float32) + b2_ref[...]


def _edge_matmul(ms, W2, b2, blk=512):
    E, d_h = ms.shape
    d_out = W2.shape[1]
    return pl.pallas_call(
        _edge_body,
        grid=(E // blk,),
        in_specs=[
            pl.BlockSpec((blk, d_h), lambda i: (i, 0)),
            pl.BlockSpec((d_h, d_out), lambda i: (0, 0)),
            pl.BlockSpec((1, d_out), lambda i: (0, 0)),
        ],
        out_specs=pl.BlockSpec((blk, d_out), lambda i: (i, 0)),
        out_shape=jax.ShapeDtypeStruct((E, d_out), jnp.float32),
    )(ms, W2, b2.reshape(1, d_out))


def _prep_sc(dst):
    @functools.partial(
        pl.kernel,
        mesh=_mesh(),
        compiler_params=pltpu.CompilerParams(needs_layout_passes=False),
        out_type=[
            jax.ShapeDtypeStruct((_NW * _CAP,), jnp.int32),
            jax.ShapeDtypeStruct((_NW * 16,), jnp.int32),
        ],
        scratch_types=[
            pltpu.VMEM((_CH,), jnp.int32),
            pltpu.VMEM((_FLUSH + 16,), jnp.int32),
            pltpu.VMEM((16,), jnp.int32),
        ],
    )
    def prep(dst_hbm, pk_hbm, cnt_hbm, dchunk, pkbuf, stg):
        wid = _wid()
        lo = wid * _BKT
        iota = lax.iota(jnp.int32, 16)
        sent = jnp.full((16,), _BKT, jnp.int32)

        def flush(off, base):
            pltpu.sync_copy(pkbuf.at[pl.ds(0, _FLUSH)],
                            pk_hbm.at[pl.ds(pl.multiple_of(wid * _CAP + base, _FLUSH), _FLUSH)])
            pkbuf[pl.ds(0, 16)] = pkbuf[pl.ds(_FLUSH, 16)]
            return off - _FLUSH, base + _FLUSH

        def maybe_flush(off, base):
            return lax.cond(off >= _FLUSH, flush, lambda o, b: (o, b), off, base)

        def chunk_body(ci, carry):
            pltpu.sync_copy(dst_hbm.at[pl.ds(pl.multiple_of(ci * _CH, _CH), _CH)], dchunk)

            def vec_body(v, carry):
                off, base = carry
                d16 = dchunk[pl.ds(v * 16, 16)]
                m = (d16 >= lo) & (d16 < lo + _BKT)
                c = plsc.all_reduce_population_count(m)[0]

                def do(off, base):
                    packed = (ci * _CH + v * 16 + iota) * 512 + (d16 - lo)
                    pk_sorted = lax.sort(
                        jnp.where(m, packed, jnp.int32(0x7FFFFFFF)), dimension=0)
                    pkbuf[pl.ds(off, 16)] = pk_sorted
                    return maybe_flush(off + c, base)

                return lax.cond(c > 0, do, lambda o, b: (o, b), off, base)

            return lax.fori_loop(0, _CH // 16, vec_body, carry)

        off, base = lax.fori_loop(0, _E // _CH, chunk_body, (0, 0))

        pad = (-off) % _B

        def pad_body(i, carry):
            off, base = carry

            def do(off, base):
                pkbuf[pl.ds(off, 16)] = sent
                return maybe_flush(off + 16, base)

            return lax.cond(i * 16 < pad, do, lambda o, b: (o, b), off, base)

        off, base = lax.fori_loop(0, _B // 16, pad_body, (off, base))
        pltpu.sync_copy(pkbuf.at[pl.ds(0, _FLUSH)],
                        pk_hbm.at[pl.ds(pl.multiple_of(wid * _CAP + base, _FLUSH), _FLUSH)])
        stg[pl.ds(0, 16)] = jnp.full((16,), 1, jnp.int32) * (base + off)
        pltpu.sync_copy(stg.at[pl.ds(0, 16)], cnt_hbm.at[pl.ds(pl.multiple_of(wid * 16, 16), 16)])

    return prep(dst)


_BG = 64
_NBG = (_E // _BG) // _NW
_RMG = _E // _BG - _NBG * _NW

def _gather_msum(U, V, src, dst):
    d_h = U.shape[1]

    @functools.partial(
        pl.kernel,
        mesh=_mesh(),
        out_type=jax.ShapeDtypeStruct((_E, d_h), jnp.float32),
        scratch_types=[
            pltpu.VMEM((_NBG * _BG + _BG,), jnp.int32),
            pltpu.VMEM((_NBG * _BG + _BG,), jnp.int32),
            pltpu.VMEM((3, _BG, d_h), jnp.float32),
            pltpu.VMEM((3, _BG, d_h), jnp.float32),
        ] + [pltpu.SemaphoreType.DMA] * 9,
    )
    def gat(u_hbm, v_hbm, src_hbm, dst_hbm, ms_hbm,
            idxd, idxs, ub, vb, gu0, gu1, gu2, gv0, gv1, gv2, s0, s1, s2):
        wid = _wid()
        gu = [gu0, gu1, gu2]
        gv = [gv0, gv1, gv2]
        ss = [s0, s1, s2]
        start = wid * _NBG
        e_lo = pl.multiple_of(start * _BG, _BG)
        pltpu.sync_copy(dst_hbm.at[pl.ds(e_lo, _NBG * _BG)],
                        idxd.at[pl.ds(0, _NBG * _BG)])
        pltpu.sync_copy(src_hbm.at[pl.ds(e_lo, _NBG * _BG)],
                        idxs.at[pl.ds(0, _NBG * _BG)])

        def uv_start(k, slot):
            pltpu.async_copy(
                u_hbm.at[idxd.at[pl.ds(k * _BG, _BG)]], ub.at[slot], gu[slot])
            pltpu.async_copy(
                v_hbm.at[idxs.at[pl.ds(k * _BG, _BG)]], vb.at[slot], gv[slot])

        def uv_wait(k, slot):
            pltpu.make_async_copy(
                u_hbm.at[idxd.at[pl.ds(k * _BG, _BG)]], ub.at[slot],
                gu[slot]).wait()
            pltpu.make_async_copy(
                v_hbm.at[idxs.at[pl.ds(k * _BG, _BG)]], vb.at[slot],
                gv[slot]).wait()

        def add_uv(slot):
            def ab(i, _):
                for j in range(d_h // 16):
                    ub[slot, i, pl.ds(j * 16, 16)] = (
                        ub[slot, i, pl.ds(j * 16, 16)]
                        + vb[slot, i, pl.ds(j * 16, 16)])
                return 0

            lax.fori_loop(0, _BG, ab, 0)

        def st_wait(k, slot):
            pltpu.make_async_copy(
                ub.at[slot], ms_hbm.at[pl.ds((start + k) * _BG, _BG)],
                ss[slot]).wait()

        uv_start(0, 0)

        def body(kk, _):
            for j in range(3):
                k = kk * 3 + j
                nslot = (j + 1) % 3

                @pl.when(k >= 2)
                def _():
                    st_wait(k - 2, nslot)

                @pl.when(k + 1 < _NBG)
                def _():
                    uv_start(k + 1, nslot)

                uv_wait(k, j)
                add_uv(j)
                pltpu.async_copy(
                    ub.at[j], ms_hbm.at[pl.ds((start + k) * _BG, _BG)], ss[j])
            return 0

        lax.fori_loop(0, _NBG // 3, body, 0)
        st_wait(_NBG - 2, (_NBG - 2) % 3)
        st_wait(_NBG - 1, (_NBG - 1) % 3)

        @pl.when(wid < _RMG)
        def _():
            e0 = pl.multiple_of((_NW * _NBG + wid) * _BG, _BG)
            pltpu.sync_copy(dst_hbm.at[pl.ds(e0, _BG)],
                            idxd.at[pl.ds(_NBG * _BG, _BG)])
            pltpu.sync_copy(src_hbm.at[pl.ds(e0, _BG)],
                            idxs.at[pl.ds(_NBG * _BG, _BG)])
            pltpu.async_copy(
                u_hbm.at[idxd.at[pl.ds(_NBG * _BG, _BG)]], ub.at[0],
                gu[0]).wait()
            pltpu.async_copy(
                v_hbm.at[idxs.at[pl.ds(_NBG * _BG, _BG)]], vb.at[0],
                gv[0]).wait()
            add_uv(0)
            pltpu.sync_copy(ub.at[0], ms_hbm.at[pl.ds(e0, _BG)])

    return gat(U, V, src, dst)


def _scatter_max(H, pk, cnts):
    d_out = H.shape[1]
    BH = 64
    CHP = 1024

    @functools.partial(
        pl.kernel,
        mesh=_mesh(),
        out_type=jax.ShapeDtypeStruct((_NP, d_out), jnp.float32),
        scratch_types=[
            pltpu.VMEM((_BKT + 1, d_out), jnp.float32),
            pltpu.VMEM((2, BH, d_out), jnp.float32),
            pltpu.VMEM((CHP + 16,), jnp.int32),
            pltpu.VMEM((2, BH), jnp.int32),
            pltpu.VMEM((16,), jnp.int32),
            pltpu.SemaphoreType.DMA,
            pltpu.SemaphoreType.DMA,
        ],
    )
    def scat(h_hbm, pk_hbm, cnt_hbm, out_hbm,
             tbl, hbuf, pkv, idxb, cntv, g0, g1):
        wid = _wid()
        gsem = [g0, g1]
        zero16 = jnp.zeros((16,), jnp.float32)

        def zb(i, _):
            for j in range(d_out // 16):
                tbl[i, pl.ds(j * 16, 16)] = zero16
            return 0

        lax.fori_loop(0, _BKT + 1, zb, 0)

        pltpu.sync_copy(cnt_hbm.at[pl.ds(pl.multiple_of(wid * 16, 16), 16)], cntv)
        n_pad = cntv[pl.ds(0, 16)][0]
        nfull = n_pad // CHP
        rb = lax.rem(n_pad, CHP) // BH

        def mk_idx(pkoff, slot):
            for j in range(BH // 16):
                idxb[slot, pl.ds(j * 16, 16)] = lax.shift_right_logical(
                    pkv[pl.ds(pkoff + j * 16, 16)], 9)

        def g_start(slot):
            pltpu.async_copy(h_hbm.at[idxb.at[slot]], hbuf.at[slot], gsem[slot])

        def g_wait(slot):
            pltpu.make_async_copy(
                h_hbm.at[idxb.at[slot]], hbuf.at[slot], gsem[slot]).wait()

        def fold(pkoff, slot):
            def eb(i, _):
                loc = pkv[pl.ds(pkoff + i, 16)][0] & 511
                for j in range(d_out // 16):
                    a = tbl[loc, pl.ds(j * 16, 16)]
                    b = hbuf[slot, i, pl.ds(j * 16, 16)]
                    tbl[loc, pl.ds(j * 16, 16)] = jnp.maximum(a, b)
                return 0

            lax.fori_loop(0, BH, eb, 0)

        def chunk(c, _):
            coff = pl.multiple_of(wid * _CAP, 8) + c * CHP
            pltpu.sync_copy(pk_hbm.at[pl.ds(coff, CHP)], pkv.at[pl.ds(0, CHP)])
            mk_idx(0, 0)
            g_start(0)
            for b in range(CHP // BH):
                slot = b % 2
                if b + 1 < CHP // BH:
                    mk_idx((b + 1) * BH, (b + 1) % 2)
                    g_start((b + 1) % 2)
                g_wait(slot)
                fold(b * BH, slot)
            return 0

        lax.fori_loop(0, nfull, chunk, 0)

        def rchunk(r, _):
            roff = pl.multiple_of(wid * _CAP, 8) + nfull * CHP + r * BH
            pltpu.sync_copy(pk_hbm.at[pl.ds(roff, BH)], pkv.at[pl.ds(0, BH)])
            mk_idx(0, 0)
            g_start(0)
            g_wait(0)
            fold(0, 0)
            return 0

        lax.fori_loop(0, rb, rchunk, 0)
        pltpu.sync_copy(tbl.at[pl.ds(0, _BKT)],
                        out_hbm.at[pl.ds(pl.multiple_of(wid * _BKT, _BKT), _BKT)])

    return scat(H, pk, cnts)


def _layer(xp, src, dst, pk, cnts, W1, b1, W2, b2):
    f = xp.shape[1]
    A = W1[:f] - W1[f:]
    B = W1[f:]
    U, V = _uv_matmul(xp, A, B, b1)
    ms = _gather_msum(U, V, src, dst)
    H = _edge_matmul(ms, W2, b2)
    return _scatter_max(H, pk, cnts)


@jax.jit
def kernel(x, edge_index, W1_1, b1_1, W2_1, b2_1, W1_2, b1_2, W2_2, b2_2,
           W1_3, b1_3, W2_3, b2_3, W1_4, b1_4, W2_4, b2_4):
    N = x.shape[0]
    src = edge_index[0]
    dst = edge_index[1]
    pk, cnts = _prep_sc(dst)
    xp = jnp.pad(x, ((0, _NP - N), (0, 0)))
    xp = _layer(xp, src, dst, pk, cnts, W1_1, b1_1, W2_1, b2_1)
    xp = _layer(xp, src, dst, pk, cnts, W1_2, b1_2, W2_2, b2_2)
    xp = _layer(xp, src, dst, pk, cnts, W1_3, b1_3, W2_3, b2_3)
    xp = _layer(xp, src, dst, pk, cnts, W1_4, b1_4, W2_4, b2_4)
    return xp[:N]


# speedup vs baseline: 1.2065x; 1.2065x over previous
import functools
import jax
import jax.numpy as jnp
from jax import lax
from jax.experimental import pallas as pl
from jax.experimental.pallas import tpu as pltpu
from jax.experimental.pallas import tpu_sc as plsc

_N = 10000
_E = 320000
_NW = 32
_BKT = 320
_NP = _NW * _BKT
_CH = 6400
_FLUSH = 2048
_CAP = 322048
_B = 128

_mesh = functools.partial(
    plsc.VectorSubcoreMesh, core_axis_name="c", subcore_axis_name="s")


def _wid():
    return lax.axis_index("s") * 2 + lax.axis_index("c")


def _pack(x):
    bf = x.astype(jnp.bfloat16)
    r = bf.reshape(x.shape[:-1] + (x.shape[-1] // 2, 2))
    return lax.bitcast_convert_type(r, jnp.float32)


def _unpack(x):
    r = lax.bitcast_convert_type(x, jnp.bfloat16)
    return r.reshape(x.shape[:-1] + (x.shape[-1] * 2,))


def _split_cols(p):
    xi = lax.bitcast_convert_type(p, jnp.uint32)
    lo = lax.bitcast_convert_type(xi << 16, jnp.float32)
    hi = lax.bitcast_convert_type(xi & jnp.uint32(0xFFFF0000), jnp.float32)
    return lo.astype(jnp.bfloat16), hi.astype(jnp.bfloat16)


def _join_cols(e, o):
    eb = lax.bitcast_convert_type(e.astype(jnp.bfloat16), jnp.uint16).astype(jnp.uint32)
    ob = lax.bitcast_convert_type(o.astype(jnp.bfloat16), jnp.uint16).astype(jnp.uint32)
    return lax.bitcast_convert_type(eb | (ob << 16), jnp.float32)


def _quarters(Wm):
    return Wm[0::2, 0::2], Wm[1::2, 0::2], Wm[0::2, 1::2], Wm[1::2, 1::2]


def _uv_matmul(xpk, A, B, b1, blk=512):
    NP, fwp = xpk.shape
    f = A.shape[0]
    fw = f // 2
    d_h = A.shape[1]
    hw = d_h // 2
    hwp = max(hw, 128)
    wspec = pl.BlockSpec((fw, hw), lambda i: (0, 0))
    bspec = pl.BlockSpec((1, hw), lambda i: (0, 0))
    bf = jnp.bfloat16
    qa = [q.astype(bf) for q in _quarters(A)]
    qb = [q.astype(bf) for q in _quarters(B)]

    def body(x_ref, aee, aoe, aeo, aoo, bee, boe, beo, boo, b1e, b1o,
             u_ref, v_ref):
        lo, hi = _split_cols(x_ref[:, :fw])

        def mm(wl, wr):
            return (jnp.dot(lo, wl[...], preferred_element_type=jnp.float32)
                    + jnp.dot(hi, wr[...], preferred_element_type=jnp.float32))

        u_ref[:, :hw] = _join_cols(mm(aee, aoe) + b1e[...], mm(aeo, aoo) + b1o[...])
        v_ref[:, :hw] = _join_cols(mm(bee, boe), mm(beo, boo))

    return pl.pallas_call(
        body,
        grid=(NP // blk,),
        in_specs=[pl.BlockSpec((blk, fwp), lambda i: (i, 0))]
        + [wspec] * 8 + [bspec] * 2,
        out_specs=[
            pl.BlockSpec((blk, hwp), lambda i: (i, 0)),
            pl.BlockSpec((blk, hwp), lambda i: (i, 0)),
        ],
        out_shape=[
            jax.ShapeDtypeStruct((NP, hwp), jnp.float32),
            jax.ShapeDtypeStruct((NP, hwp), jnp.float32),
        ],
    )(xpk, *qa, *qb, b1[0::2].reshape(1, hw), b1[1::2].reshape(1, hw))


def _edge_matmul(ms, W2, b2, blk=512):
    E, hwp = ms.shape
    d_h = W2.shape[0]
    hw = d_h // 2
    d_out = W2.shape[1]
    ow = d_out // 2
    owp = max(ow, 128)
    wspec = pl.BlockSpec((hw, ow), lambda i: (0, 0))
    bspec = pl.BlockSpec((1, ow), lambda i: (0, 0))
    qw = [q.astype(jnp.bfloat16) for q in _quarters(W2)]

    def body(ms_ref, wee, woe, weo, woo, b2e, b2o, h_ref):
        lo, hi = _split_cols(ms_ref[:, :hw])
        lo = jax.nn.relu(lo)
        hi = jax.nn.relu(hi)

        def mm(wl, wr):
            return (jnp.dot(lo, wl[...], preferred_element_type=jnp.float32)
                    + jnp.dot(hi, wr[...], preferred_element_type=jnp.float32))

        h_ref[:, :ow] = _join_cols(mm(wee, woe) + b2e[...], mm(weo, woo) + b2o[...])

    return pl.pallas_call(
        body,
        grid=(E // blk,),
        in_specs=[pl.BlockSpec((blk, hwp), lambda i: (i, 0))]
        + [wspec] * 4 + [bspec] * 2,
        out_specs=pl.BlockSpec((blk, owp), lambda i: (i, 0)),
        out_shape=jax.ShapeDtypeStruct((E, owp), jnp.float32),
    )(ms, *qw, b2[0::2].reshape(1, ow), b2[1::2].reshape(1, ow))


def _prep_sc(dst):
    @functools.partial(
        pl.kernel,
        mesh=_mesh(),
        compiler_params=pltpu.CompilerParams(needs_layout_passes=False),
        out_type=[
            jax.ShapeDtypeStruct((_NW * _CAP,), jnp.int32),
            jax.ShapeDtypeStruct((_NW * 16,), jnp.int32),
        ],
        scratch_types=[
            pltpu.VMEM((_CH,), jnp.int32),
            pltpu.VMEM((_FLUSH + 16,), jnp.int32),
            pltpu.VMEM((16,), jnp.int32),
        ],
    )
    def prep(dst_hbm, pk_hbm, cnt_hbm, dchunk, pkbuf, stg):
        wid = _wid()
        lo = wid * _BKT
        iota = lax.iota(jnp.int32, 16)
        sent = jnp.full((16,), _BKT, jnp.int32)

        def flush(off, base):
            pltpu.sync_copy(pkbuf.at[pl.ds(0, _FLUSH)],
                            pk_hbm.at[pl.ds(pl.multiple_of(wid * _CAP + base, _FLUSH), _FLUSH)])
            pkbuf[pl.ds(0, 16)] = pkbuf[pl.ds(_FLUSH, 16)]
            return off - _FLUSH, base + _FLUSH

        def maybe_flush(off, base):
            return lax.cond(off >= _FLUSH, flush, lambda o, b: (o, b), off, base)

        def chunk_body(ci, carry):
            pltpu.sync_copy(dst_hbm.at[pl.ds(pl.multiple_of(ci * _CH, _CH), _CH)], dchunk)

            def vec_body(v, carry):
                off, base = carry
                d16 = dchunk[pl.ds(v * 16, 16)]
                m = (d16 >= lo) & (d16 < lo + _BKT)
                packed = (ci * _CH + v * 16 + iota) * 512 + (d16 - lo)
                pk_sorted = lax.sort(
                    jnp.where(m, packed, jnp.int32(0x7FFFFFFF)), dimension=0)
                pkbuf[pl.ds(off, 16)] = pk_sorted
                off = off + jnp.sum(m.astype(jnp.int32))
                return maybe_flush(off, base)

            return lax.fori_loop(0, _CH // 16, vec_body, carry)

        off, base = lax.fori_loop(0, _E // _CH, chunk_body, (0, 0))

        pad = (-off) % _B

        def pad_body(i, carry):
            off, base = carry

            def do(off, base):
                pkbuf[pl.ds(off, 16)] = sent
                return maybe_flush(off + 16, base)

            return lax.cond(i * 16 < pad, do, lambda o, b: (o, b), off, base)

        off, base = lax.fori_loop(0, _B // 16, pad_body, (off, base))
        pltpu.sync_copy(pkbuf.at[pl.ds(0, _FLUSH)],
                        pk_hbm.at[pl.ds(pl.multiple_of(wid * _CAP + base, _FLUSH), _FLUSH)])
        stg[pl.ds(0, 16)] = jnp.full((16,), 1, jnp.int32) * (base + off)
        pltpu.sync_copy(stg.at[pl.ds(0, 16)], cnt_hbm.at[pl.ds(pl.multiple_of(wid * 16, 16), 16)])

    return prep(dst)


_BG = 64
_NBG = (_E // _BG) // _NW
_RMG = _E // _BG - _NBG * _NW

def _gather_msum(U, V, src, dst):
    W = U.shape[1]

    @functools.partial(
        pl.kernel,
        mesh=_mesh(),
        compiler_params=pltpu.CompilerParams(needs_layout_passes=False),
        out_type=jax.ShapeDtypeStruct((_E, W), jnp.float32),
        scratch_types=[
            pltpu.VMEM((_NBG * _BG + _BG,), jnp.int32),
            pltpu.VMEM((_NBG * _BG + _BG,), jnp.int32),
            pltpu.VMEM((3, _BG, W), jnp.float32),
            pltpu.VMEM((3, _BG, W), jnp.float32),
        ] + [pltpu.SemaphoreType.DMA] * 9,
    )
    def gat(u_hbm, v_hbm, src_hbm, dst_hbm, ms_hbm,
            idxd, idxs, ub, vb, gu0, gu1, gu2, gv0, gv1, gv2, s0, s1, s2):
        wid = _wid()
        gu = [gu0, gu1, gu2]
        gv = [gv0, gv1, gv2]
        ss = [s0, s1, s2]
        start = wid * _NBG
        e_lo = pl.multiple_of(start * _BG, _BG)
        pltpu.sync_copy(dst_hbm.at[pl.ds(e_lo, _NBG * _BG)],
                        idxd.at[pl.ds(0, _NBG * _BG)])
        pltpu.sync_copy(src_hbm.at[pl.ds(e_lo, _NBG * _BG)],
                        idxs.at[pl.ds(0, _NBG * _BG)])

        def uv_start(k, slot):
            pltpu.async_copy(
                u_hbm.at[idxd.at[pl.ds(k * _BG, _BG)]], ub.at[slot], gu[slot])
            pltpu.async_copy(
                v_hbm.at[idxs.at[pl.ds(k * _BG, _BG)]], vb.at[slot], gv[slot])

        def uv_wait(k, slot):
            pltpu.make_async_copy(
                u_hbm.at[idxd.at[pl.ds(k * _BG, _BG)]], ub.at[slot],
                gu[slot]).wait()
            pltpu.make_async_copy(
                v_hbm.at[idxs.at[pl.ds(k * _BG, _BG)]], vb.at[slot],
                gv[slot]).wait()

        def add_uv(slot):
            def ab(i, _):
                for j in range(W // 16):
                    a = plsc.bitcast(ub[slot, i, pl.ds(j * 16, 16)], jnp.bfloat16)
                    b = plsc.bitcast(vb[slot, i, pl.ds(j * 16, 16)], jnp.bfloat16)
                    ub[slot, i, pl.ds(j * 16, 16)] = plsc.bitcast(
                        a + b, jnp.float32)
                return 0

            lax.fori_loop(0, _BG, ab, 0)

        def st_wait(k, slot):
            pltpu.make_async_copy(
                ub.at[slot], ms_hbm.at[pl.ds((start + k) * _BG, _BG)],
                ss[slot]).wait()

        uv_start(0, 0)

        def body(kk, _):
            for j in range(3):
                k = kk * 3 + j
                nslot = (j + 1) % 3

                @pl.when(k >= 2)
                def _():
                    st_wait(k - 2, nslot)

                @pl.when(k + 1 < _NBG)
                def _():
                    uv_start(k + 1, nslot)

                uv_wait(k, j)
                add_uv(j)
                pltpu.async_copy(
                    ub.at[j], ms_hbm.at[pl.ds((start + k) * _BG, _BG)], ss[j])
            return 0

        lax.fori_loop(0, _NBG // 3, body, 0)
        st_wait(_NBG - 2, (_NBG - 2) % 3)
        st_wait(_NBG - 1, (_NBG - 1) % 3)

        @pl.when(wid < _RMG)
        def _():
            e0 = pl.multiple_of((_NW * _NBG + wid) * _BG, _BG)
            pltpu.sync_copy(dst_hbm.at[pl.ds(e0, _BG)],
                            idxd.at[pl.ds(_NBG * _BG, _BG)])
            pltpu.sync_copy(src_hbm.at[pl.ds(e0, _BG)],
                            idxs.at[pl.ds(_NBG * _BG, _BG)])
            pltpu.async_copy(
                u_hbm.at[idxd.at[pl.ds(_NBG * _BG, _BG)]], ub.at[0],
                gu[0]).wait()
            pltpu.async_copy(
                v_hbm.at[idxs.at[pl.ds(_NBG * _BG, _BG)]], vb.at[0],
                gv[0]).wait()
            add_uv(0)
            pltpu.sync_copy(ub.at[0], ms_hbm.at[pl.ds(e0, _BG)])

    return gat(U, V, src, dst)


def _scatter_max(H, pk, cnts):
    W = H.shape[1]
    BH = 64
    CHP = 1024

    @functools.partial(
        pl.kernel,
        mesh=_mesh(),
        compiler_params=pltpu.CompilerParams(needs_layout_passes=False),
        out_type=jax.ShapeDtypeStruct((_NP, W), jnp.float32),
        scratch_types=[
            pltpu.VMEM((_BKT + 1, W), jnp.float32),
            pltpu.VMEM((2, BH, W), jnp.float32),
            pltpu.VMEM((CHP + 16,), jnp.int32),
            pltpu.VMEM((2, BH), jnp.int32),
            pltpu.VMEM((16,), jnp.int32),
            pltpu.SemaphoreType.DMA,
            pltpu.SemaphoreType.DMA,
        ],
    )
    def scat(h_hbm, pk_hbm, cnt_hbm, out_hbm,
             tbl, hbuf, pkv, idxb, cntv, g0, g1):
        wid = _wid()
        gsem = [g0, g1]
        zero16 = jnp.zeros((16,), jnp.float32)

        def zb(i, _):
            for j in range(W // 16):
                tbl[i, pl.ds(j * 16, 16)] = zero16
            return 0

        lax.fori_loop(0, _BKT + 1, zb, 0)

        pltpu.sync_copy(cnt_hbm.at[pl.ds(pl.multiple_of(wid * 16, 16), 16)], cntv)
        n_pad = cntv[pl.ds(0, 16)][0]
        nfull = n_pad // CHP
        rb = lax.rem(n_pad, CHP) // BH

        def mk_idx(pkoff, slot):
            for j in range(BH // 16):
                idxb[slot, pl.ds(j * 16, 16)] = lax.shift_right_logical(
                    pkv[pl.ds(pkoff + j * 16, 16)], 9)

        def g_start(slot):
            pltpu.async_copy(h_hbm.at[idxb.at[slot]], hbuf.at[slot], gsem[slot])

        def g_wait(slot):
            pltpu.make_async_copy(
                h_hbm.at[idxb.at[slot]], hbuf.at[slot], gsem[slot]).wait()

        def fold(pkoff, slot):
            def eb(i, _):
                loc = pkv[pl.ds(pkoff + i, 16)][0] & 511
                for j in range(W // 16):
                    a = plsc.bitcast(tbl[loc, pl.ds(j * 16, 16)], jnp.bfloat16)
                    b = plsc.bitcast(hbuf[slot, i, pl.ds(j * 16, 16)], jnp.bfloat16)
                    tbl[loc, pl.ds(j * 16, 16)] = plsc.bitcast(
                        jnp.maximum(a, b), jnp.float32)
                return 0

            lax.fori_loop(0, BH, eb, 0)

        def chunk(c, _):
            coff = pl.multiple_of(wid * _CAP, 8) + c * CHP
            pltpu.sync_copy(pk_hbm.at[pl.ds(coff, CHP)], pkv.at[pl.ds(0, CHP)])
            mk_idx(0, 0)
            g_start(0)
            for b in range(CHP // BH):
                slot = b % 2
                if b + 1 < CHP // BH:
                    mk_idx((b + 1) * BH, (b + 1) % 2)
                    g_start((b + 1) % 2)
                g_wait(slot)
                fold(b * BH, slot)
            return 0

        lax.fori_loop(0, nfull, chunk, 0)

        def rchunk(r, _):
            roff = pl.multiple_of(wid * _CAP, 8) + nfull * CHP + r * BH
            pltpu.sync_copy(pk_hbm.at[pl.ds(roff, BH)], pkv.at[pl.ds(0, BH)])
            mk_idx(0, 0)
            g_start(0)
            g_wait(0)
            fold(0, 0)
            return 0

        lax.fori_loop(0, rb, rchunk, 0)
        pltpu.sync_copy(tbl.at[pl.ds(0, _BKT)],
                        out_hbm.at[pl.ds(pl.multiple_of(wid * _BKT, _BKT), _BKT)])

    return scat(H, pk, cnts)


def _layer(xpk, src, dst, pk, cnts, W1, b1, W2, b2):
    f = W1.shape[0] // 2
    A = W1[:f] - W1[f:]
    B = W1[f:]
    U, V = _uv_matmul(xpk, A, B, b1)
    ms = _gather_msum(U, V, src, dst)
    H = _edge_matmul(ms, W2, b2)
    return _scatter_max(H, pk, cnts)


@jax.jit
def kernel(x, edge_index, W1_1, b1_1, W2_1, b2_1, W1_2, b1_2, W2_2, b2_2,
           W1_3, b1_3, W2_3, b2_3, W1_4, b1_4, W2_4, b2_4):
    N = x.shape[0]
    src = edge_index[0]
    dst = edge_index[1]
    pk, cnts = _prep_sc(dst)
    xpk = _pack(jnp.pad(x, ((0, _NP - N), (0, 0))))
    xpk = jnp.pad(xpk, ((0, 0), (0, 128 - xpk.shape[1])))
    xpk = _layer(xpk, src, dst, pk, cnts, W1_1, b1_1, W2_1, b2_1)
    xpk = _layer(xpk, src, dst, pk, cnts, W1_2, b1_2, W2_2, b2_2)
    xpk = _layer(xpk, src, dst, pk, cnts, W1_3, b1_3, W2_3, b2_3)
    xpk = _layer(xpk, src, dst, pk, cnts, W1_4, b1_4, W2_4, b2_4)
    return _unpack(xpk[:, :x.shape[1] // 2])[:N].astype(jnp.float32)
